# Initial kernel scaffold; baseline (speedup 1.0000x reference)
import jax
import jax.numpy as jnp
from jax.experimental import pallas as pl


def kernel(inputs, adj, W_l, W_r, b):
    raise NotImplementedError("write your pallas kernel here")


# speedup vs baseline: 1.0415x; 1.0415x over previous
import functools

import jax
import jax.numpy as jnp
from jax import lax
from jax.experimental import pallas as pl
from jax.experimental.pallas import tpu as pltpu
from jax.experimental.pallas import tpu_sc as plsc

NB = 2
NN = 10000
NE = 320000
D = 128

NC = 2
NS = 16
NW = NC * NS
NP = D // 16
CHUNK = 80
EPT = NE // NS
NCHUNK = EPT // CHUNK
CPT = NE // NW
NCCHUNK = CPT // CHUNK
NNP = 10240
RPT = NNP // NS
CW = 16
BLK = 2000


def _sc_body(src, dst, xr, acc_out, cnt_out,
             src_v, dst_v, rows_v, ones_v, zbuf_v, acc_sh, sem):
    c = lax.axis_index("c")
    s = lax.axis_index("s")
    rbase = s * RPT

    zero16 = jnp.zeros((16,), jnp.float32)
    one16 = jnp.ones((16,), jnp.float32)

    def fill_zbuf(i, _):
        zbuf_v[i, pl.ds(0, 16)] = zero16
        return 0
    lax.fori_loop(0, RPT, fill_zbuf, 0)

    def fill_ones(i, _):
        ones_v[i, pl.ds(0, 16)] = one16
        return 0
    lax.fori_loop(0, CHUNK, fill_ones, 0)

    pltpu.sync_copy(zbuf_v, acc_sh.at[pl.ds(rbase, RPT)])
    plsc.subcore_barrier()

    def cbody(j, _):
        off = (c * NS + s) * CPT + j * CHUNK
        pltpu.sync_copy(dst.at[pl.ds(off, CHUNK)], dst_v)
        pltpu.sync_copy(ones_v, acc_sh.at[dst_v], add=True)
        return 0
    lax.fori_loop(0, NCCHUNK, cbody, 0)
    plsc.subcore_barrier()
    pltpu.sync_copy(acc_sh.at[pl.ds(rbase, RPT)],
                    cnt_out.at[c, pl.ds(rbase, RPT)])
    pltpu.sync_copy(zbuf_v, acc_sh.at[pl.ds(rbase, RPT)])
    plsc.subcore_barrier()

    for p in range(NP):
        toff = (c * NP + p) * NN

        def body(j, _):
            off = s * EPT + j * CHUNK
            pltpu.sync_copy(src.at[pl.ds(off, CHUNK)], src_v)
            for k in range(CHUNK // 16):
                src_v[pl.ds(k * 16, 16)] = src_v[pl.ds(k * 16, 16)] + toff
            pltpu.sync_copy(dst.at[pl.ds(off, CHUNK)], dst_v)
            pltpu.async_copy(xr.at[src_v], rows_v, sem).wait()
            pltpu.sync_copy(rows_v, acc_sh.at[dst_v], add=True)
            return 0
        lax.fori_loop(0, NCHUNK, body, 0)
        plsc.subcore_barrier()

        pltpu.sync_copy(acc_sh.at[pl.ds(rbase, RPT)],
                        acc_out.at[c, p, pl.ds(rbase, RPT)])
        if p + 1 < NP:
            pltpu.sync_copy(zbuf_v, acc_sh.at[pl.ds(rbase, RPT)])
            plsc.subcore_barrier()


_sc_segment_sum = functools.partial(
    pl.kernel,
    out_type=(
        jax.ShapeDtypeStruct((NB, NP, NNP, CW), jnp.float32),
        jax.ShapeDtypeStruct((NC, NNP, CW), jnp.float32),
    ),
    mesh=plsc.VectorSubcoreMesh(core_axis_name="c", subcore_axis_name="s"),
    compiler_params=pltpu.CompilerParams(use_tc_tiling_on_sc=False),
    scratch_types=[
        pltpu.VMEM((CHUNK,), jnp.int32),
        pltpu.VMEM((CHUNK,), jnp.int32),
        pltpu.VMEM((CHUNK, CW), jnp.float32),
        pltpu.VMEM((CHUNK, CW), jnp.float32),
        pltpu.VMEM((RPT, CW), jnp.float32),
        pltpu.VMEM_SHARED((NNP, CW), jnp.float32),
        pltpu.SemaphoreType.DMA,
    ],
)(_sc_body)


def _tc_body(x_ref, acc_ref, cnt_ref, wl_ref, wr_ref, b_ref, o_ref):
    mean_cat = jnp.concatenate([acc_ref[0, p] for p in range(NP)], axis=-1)
    cnt = cnt_ref[0, :, :1] + cnt_ref[1, :, :1]
    mean = mean_cat / jnp.maximum(cnt, 1.0)
    o = jnp.dot(mean, wl_ref[...], preferred_element_type=jnp.float32)
    o = o + jnp.dot(x_ref[0], wr_ref[...], preferred_element_type=jnp.float32)
    o = o + b_ref[...]
    o_ref[0] = jnp.maximum(o, 0.0)


def _tc_tail(x, acc, cnt, W_l, W_r, b2):
    grid = (NB, NN // BLK)
    return pl.pallas_call(
        _tc_body,
        grid=grid,
        in_specs=[
            pl.BlockSpec((1, BLK, D), lambda i, j: (i, j, 0)),
            pl.BlockSpec((1, NP, BLK, CW), lambda i, j: (i, 0, j, 0)),
            pl.BlockSpec((NC, BLK, CW), lambda i, j: (0, j, 0)),
            pl.BlockSpec((D, D), lambda i, j: (0, 0)),
            pl.BlockSpec((D, D), lambda i, j: (0, 0)),
            pl.BlockSpec((1, D), lambda i, j: (0, 0)),
        ],
        out_specs=pl.BlockSpec((1, BLK, D), lambda i, j: (i, j, 0)),
        out_shape=jax.ShapeDtypeStruct((NB, NN, D), jnp.float32),
    )(x, acc, cnt, W_l, W_r, b2)


def kernel(inputs, adj, W_l, W_r, b):
    x = inputs
    xr = x.reshape(NB, NN, NP, CW).transpose(0, 2, 1, 3).reshape(-1, CW)
    acc, cnt = _sc_segment_sum(adj[0], adj[1], xr)
    return _tc_tail(x, acc, cnt, W_l, W_r, b.reshape(1, D))


# speedup vs baseline: 3.4039x; 3.2683x over previous
import functools

import jax
import jax.numpy as jnp
from jax import lax
from jax.experimental import pallas as pl
from jax.experimental.pallas import tpu as pltpu
from jax.experimental.pallas import tpu_sc as plsc

NB = 2
NN = 10000
NE = 320000
D = 128

NC = 2
NS = 16
NW = NC * NS
NP = D // 16
NG = NB * NP
CHUNK = 80
EPT = NE // NS
NCHUNK = EPT // CHUNK
CPT = NE // NW
NCCHUNK = CPT // CHUNK
NNP = 10240
RPT = NNP // NS
CW = 16
BLK = 2000


def _sc_body(srcb, dst, xr, acc_out, cnt_out,
             src_v0, src_v1, src_v2, src_v3,
             dst_v0, dst_v1, dst_v2, dst_v3,
             rows_v0, rows_v1, ones_v, zbuf_v, acc_sh,
             si0, si1, si2, si3, sg0, sg1, ss0, ss1):
    src_v = [src_v0, src_v1, src_v2, src_v3]
    dst_v = [dst_v0, dst_v1, dst_v2, dst_v3]
    rows_v = [rows_v0, rows_v1]
    si = [si0, si1, si2, si3]
    sg = [sg0, sg1]
    ss = [ss0, ss1]

    c = lax.axis_index("c")
    s = lax.axis_index("s")
    rbase = s * RPT

    zero16 = jnp.zeros((16,), jnp.float32)
    one16 = jnp.ones((16,), jnp.float32)

    def fill_zbuf(i, _):
        zbuf_v[i, pl.ds(0, 16)] = zero16
        return 0
    lax.fori_loop(0, RPT, fill_zbuf, 0)

    def fill_ones(i, _):
        ones_v[i, pl.ds(0, 16)] = one16
        return 0
    lax.fori_loop(0, CHUNK, fill_ones, 0)

    pltpu.sync_copy(zbuf_v, acc_sh.at[pl.ds(rbase, RPT)])
    plsc.subcore_barrier()

    def cnt_fire_idx(j, q):
        off = (c * NS + s) * CPT + j * CHUNK
        pltpu.async_copy(dst.at[pl.ds(off, CHUNK)], dst_v[q], si[q])

    def cnt_visit(j, r, dsc, dfi):
        q, k, q2 = r % 4, r % 2, (r - 2) % 4
        pltpu.make_async_copy(dst.at[pl.ds(0, CHUNK)], dst_v[q], si[q]).wait()
        if dsc:
            pltpu.make_async_copy(ones_v, acc_sh.at[dst_v[q2]], ss[k]).wait()
        pltpu.async_copy(ones_v, acc_sh.at[dst_v[q]], ss[k], add=True)
        if dfi:
            cnt_fire_idx(j + 2, (r + 2) % 4)

    cnt_fire_idx(0, 0)
    cnt_fire_idx(1, 1)
    cnt_visit(0, 0, False, True)
    cnt_visit(1, 1, False, True)
    cnt_visit(2, 2, True, True)
    cnt_visit(3, 3, True, True)

    def cbody(i, _):
        for r in range(4):
            cnt_visit(4 * i + r, r, True, True)
        return 0
    lax.fori_loop(1, 30, cbody, 0)
    for j in range(120, NCCHUNK):
        cnt_visit(j, j % 4, True, j + 2 < NCCHUNK)
    pltpu.make_async_copy(ones_v, acc_sh.at[dst_v[3]], ss[1]).wait()
    pltpu.make_async_copy(ones_v, acc_sh.at[dst_v[0]], ss[0]).wait()

    plsc.subcore_barrier()
    pltpu.sync_copy(acc_sh.at[pl.ds(rbase, RPT)],
                    cnt_out.at[c, pl.ds(rbase, RPT)])
    pltpu.sync_copy(zbuf_v, acc_sh.at[pl.ds(rbase, RPT)])
    plsc.subcore_barrier()

    for p in range(NP):
        g = c * NP + p

        def fire_idx(j, q):
            off = s * EPT + j * CHUNK
            pltpu.async_copy(srcb.at[g, pl.ds(off, CHUNK)], src_v[q], si[q])
            pltpu.async_copy(dst.at[pl.ds(off, CHUNK)], dst_v[q], si[q])

        def col_visit(j, r, dsc, dscp, dfi):
            q, k = r % 4, r % 2
            qm, km = (r - 1) % 4, (r - 1) % 2
            q2 = (r - 2) % 4
            pltpu.make_async_copy(srcb.at[0, pl.ds(0, CHUNK)],
                                  src_v[q], si[q]).wait()
            pltpu.make_async_copy(dst.at[pl.ds(0, CHUNK)],
                                  dst_v[q], si[q]).wait()
            if dsc:
                pltpu.make_async_copy(rows_v[k], acc_sh.at[dst_v[q2]],
                                      ss[k]).wait()
            pltpu.async_copy(xr.at[src_v[q]], rows_v[k], sg[k])
            if dscp:
                pltpu.make_async_copy(xr.at[pl.ds(0, CHUNK)],
                                      rows_v[km], sg[km]).wait()
                pltpu.async_copy(rows_v[km], acc_sh.at[dst_v[qm]],
                                 ss[km], add=True)
            if dfi:
                fire_idx(j + 2, (r + 2) % 4)

        fire_idx(0, 0)
        fire_idx(1, 1)
        col_visit(0, 0, False, False, True)
        col_visit(1, 1, False, True, True)
        col_visit(2, 2, True, True, True)
        col_visit(3, 3, True, True, True)

        def body(i, _):
            for r in range(4):
                col_visit(4 * i + r, r, True, True, True)
            return 0
        lax.fori_loop(1, NCHUNK // 4, body, 0)
        for j in range(NCHUNK - 2, NCHUNK):
            col_visit(j, j % 4, True, True, False)
        pltpu.make_async_copy(xr.at[pl.ds(0, CHUNK)], rows_v[1], sg[1]).wait()
        pltpu.async_copy(rows_v[1], acc_sh.at[dst_v[1]], ss[1], add=True)
        pltpu.make_async_copy(rows_v[0], acc_sh.at[dst_v[0]], ss[0]).wait()
        pltpu.make_async_copy(rows_v[1], acc_sh.at[dst_v[1]], ss[1]).wait()

        plsc.subcore_barrier()
        pltpu.sync_copy(acc_sh.at[pl.ds(rbase, RPT)],
                        acc_out.at[c, p, pl.ds(rbase, RPT)])
        if p + 1 < NP:
            pltpu.sync_copy(zbuf_v, acc_sh.at[pl.ds(rbase, RPT)])
            plsc.subcore_barrier()


_sc_segment_sum = functools.partial(
    pl.kernel,
    out_type=(
        jax.ShapeDtypeStruct((NB, NP, NNP, CW), jnp.float32),
        jax.ShapeDtypeStruct((NC, NNP, CW), jnp.float32),
    ),
    mesh=plsc.VectorSubcoreMesh(core_axis_name="c", subcore_axis_name="s"),
    compiler_params=pltpu.CompilerParams(use_tc_tiling_on_sc=False),
    scratch_types=[
        pltpu.VMEM((CHUNK,), jnp.int32),
        pltpu.VMEM((CHUNK,), jnp.int32),
        pltpu.VMEM((CHUNK,), jnp.int32),
        pltpu.VMEM((CHUNK,), jnp.int32),
        pltpu.VMEM((CHUNK,), jnp.int32),
        pltpu.VMEM((CHUNK,), jnp.int32),
        pltpu.VMEM((CHUNK,), jnp.int32),
        pltpu.VMEM((CHUNK,), jnp.int32),
        pltpu.VMEM((CHUNK, CW), jnp.float32),
        pltpu.VMEM((CHUNK, CW), jnp.float32),
        pltpu.VMEM((CHUNK, CW), jnp.float32),
        pltpu.VMEM((RPT, CW), jnp.float32),
        pltpu.VMEM_SHARED((NNP, CW), jnp.float32),
        pltpu.SemaphoreType.DMA,
        pltpu.SemaphoreType.DMA,
        pltpu.SemaphoreType.DMA,
        pltpu.SemaphoreType.DMA,
        pltpu.SemaphoreType.DMA,
        pltpu.SemaphoreType.DMA,
        pltpu.SemaphoreType.DMA,
        pltpu.SemaphoreType.DMA,
    ],
)(_sc_body)


def _tc_body(x_ref, acc_ref, cnt_ref, wl_ref, wr_ref, b_ref, o_ref):
    mean_cat = jnp.concatenate([acc_ref[0, p] for p in range(NP)], axis=-1)
    cnt = cnt_ref[0, :, :1] + cnt_ref[1, :, :1]
    mean = mean_cat / jnp.maximum(cnt, 1.0)
    o = jnp.dot(mean, wl_ref[...], preferred_element_type=jnp.float32)
    o = o + jnp.dot(x_ref[0], wr_ref[...], preferred_element_type=jnp.float32)
    o = o + b_ref[...]
    o_ref[0] = jnp.maximum(o, 0.0)


def _tc_tail(x, acc, cnt, W_l, W_r, b2):
    grid = (NB, NN // BLK)
    return pl.pallas_call(
        _tc_body,
        grid=grid,
        in_specs=[
            pl.BlockSpec((1, BLK, D), lambda i, j: (i, j, 0)),
            pl.BlockSpec((1, NP, BLK, CW), lambda i, j: (i, 0, j, 0)),
            pl.BlockSpec((NC, BLK, CW), lambda i, j: (0, j, 0)),
            pl.BlockSpec((D, D), lambda i, j: (0, 0)),
            pl.BlockSpec((D, D), lambda i, j: (0, 0)),
            pl.BlockSpec((1, D), lambda i, j: (0, 0)),
        ],
        out_specs=pl.BlockSpec((1, BLK, D), lambda i, j: (i, j, 0)),
        out_shape=jax.ShapeDtypeStruct((NB, NN, D), jnp.float32),
    )(x, acc, cnt, W_l, W_r, b2)


def kernel(inputs, adj, W_l, W_r, b):
    x = inputs
    xr = x.reshape(NB, NN, NP, CW).transpose(0, 2, 1, 3).reshape(-1, CW)
    src = adj[0]
    srcb = src[None, :] + (jnp.arange(NG, dtype=jnp.int32) * NN)[:, None]
    acc, cnt = _sc_segment_sum(srcb, adj[1], xr)
    return _tc_tail(x, acc, cnt, W_l, W_r, b.reshape(1, D))


# speedup vs baseline: 3.4496x; 1.0134x over previous
import functools

import jax
import jax.numpy as jnp
from jax import lax
from jax.experimental import pallas as pl
from jax.experimental.pallas import tpu as pltpu
from jax.experimental.pallas import tpu_sc as plsc

NB = 2
NN = 10000
NE = 320000
D = 128

NC = 2
NS = 16
NW = NC * NS
NP = D // 16
NG = NB * NP
CHUNK = 80
EPT = NE // NS
NCHUNK = EPT // CHUNK
NCCHUNK = NCHUNK // NC
NNP = 10240
RPT = NNP // NS
CW = 16
BLK = 2000


def _sc_body(srcb, dstb, xr, acc_out, cnt_out,
             src_st, dst_st, rows_v0, rows_v1, ones_v, zbuf_v, acc_sh,
             sg0, sg1, ss0, ss1):
    rows_v = [rows_v0, rows_v1]
    sg = [sg0, sg1]
    ss = [ss0, ss1]

    c = lax.axis_index("c")
    s = lax.axis_index("s")
    rbase = s * RPT

    zero16 = jnp.zeros((16,), jnp.float32)
    one16 = jnp.ones((16,), jnp.float32)

    def fill_zbuf(i, _):
        zbuf_v[i, pl.ds(0, 16)] = zero16
        return 0
    lax.fori_loop(0, RPT, fill_zbuf, 0)

    def fill_ones(i, _):
        ones_v[i, pl.ds(0, 16)] = one16
        return 0
    lax.fori_loop(0, CHUNK, fill_ones, 0)

    pltpu.sync_copy(dstb.at[s], dst_st)
    pltpu.sync_copy(zbuf_v, acc_sh.at[pl.ds(rbase, RPT)])
    plsc.subcore_barrier()

    def cvisit(j, k, dsc):
        row = c * NCCHUNK + j
        if dsc:
            pltpu.make_async_copy(ones_v, acc_sh.at[dst_st.at[row - 2]],
                                  ss[k]).wait()
        pltpu.async_copy(ones_v, acc_sh.at[dst_st.at[row]], ss[k], add=True)

    cvisit(0, 0, False)
    cvisit(1, 1, False)

    def cbody(i, _):
        for r in range(2):
            cvisit(2 * i + r, r, True)
        return 0
    lax.fori_loop(1, NCCHUNK // 2, cbody, 0)
    cvisit(NCCHUNK - 1, 0, True)
    pltpu.make_async_copy(ones_v, acc_sh.at[dst_st.at[c * NCCHUNK +
                                                      NCCHUNK - 2]],
                          ss[1]).wait()
    pltpu.make_async_copy(ones_v, acc_sh.at[dst_st.at[c * NCCHUNK +
                                                      NCCHUNK - 1]],
                          ss[0]).wait()

    plsc.subcore_barrier()
    pltpu.sync_copy(acc_sh.at[pl.ds(rbase, RPT)],
                    cnt_out.at[c, pl.ds(rbase, RPT)])
    pltpu.sync_copy(zbuf_v, acc_sh.at[pl.ds(rbase, RPT)])
    plsc.subcore_barrier()

    for p in range(NP):
        g = c * NP + p
        pltpu.sync_copy(srcb.at[g, s], src_st)

        def visit(j, r, dsc, dscp):
            k, km = r % 2, (r - 1) % 2
            if dsc:
                pltpu.make_async_copy(rows_v[k], acc_sh.at[dst_st.at[j - 2]],
                                      ss[k]).wait()
            pltpu.async_copy(xr.at[src_st.at[j]], rows_v[k], sg[k])
            if dscp:
                pltpu.make_async_copy(xr.at[pl.ds(0, CHUNK)], rows_v[km],
                                      sg[km]).wait()
                pltpu.async_copy(rows_v[km], acc_sh.at[dst_st.at[j - 1]],
                                 ss[km], add=True)

        visit(0, 0, False, False)
        visit(1, 1, False, True)

        def body(i, _):
            for r in range(2):
                visit(2 * i + r, r, True, True)
            return 0
        lax.fori_loop(1, NCHUNK // 2, body, 0)
        pltpu.make_async_copy(xr.at[pl.ds(0, CHUNK)], rows_v[1], sg[1]).wait()
        pltpu.async_copy(rows_v[1], acc_sh.at[dst_st.at[NCHUNK - 1]],
                         ss[1], add=True)
        pltpu.make_async_copy(rows_v[0], acc_sh.at[dst_st.at[NCHUNK - 2]],
                              ss[0]).wait()
        pltpu.make_async_copy(rows_v[1], acc_sh.at[dst_st.at[NCHUNK - 1]],
                              ss[1]).wait()

        plsc.subcore_barrier()
        pltpu.sync_copy(acc_sh.at[pl.ds(rbase, RPT)],
                        acc_out.at[c, p, pl.ds(rbase, RPT)])
        if p + 1 < NP:
            pltpu.sync_copy(zbuf_v, acc_sh.at[pl.ds(rbase, RPT)])
            plsc.subcore_barrier()


_sc_segment_sum = functools.partial(
    pl.kernel,
    out_type=(
        jax.ShapeDtypeStruct((NB, NP, NNP, CW), jnp.float32),
        jax.ShapeDtypeStruct((NC, NNP, CW), jnp.float32),
    ),
    mesh=plsc.VectorSubcoreMesh(core_axis_name="c", subcore_axis_name="s"),
    compiler_params=pltpu.CompilerParams(use_tc_tiling_on_sc=False),
    scratch_types=[
        pltpu.VMEM((NCHUNK, CHUNK), jnp.int32),
        pltpu.VMEM((NCHUNK, CHUNK), jnp.int32),
        pltpu.VMEM((CHUNK, CW), jnp.float32),
        pltpu.VMEM((CHUNK, CW), jnp.float32),
        pltpu.VMEM((CHUNK, CW), jnp.float32),
        pltpu.VMEM((RPT, CW), jnp.float32),
        pltpu.VMEM_SHARED((NNP, CW), jnp.float32),
        pltpu.SemaphoreType.DMA,
        pltpu.SemaphoreType.DMA,
        pltpu.SemaphoreType.DMA,
        pltpu.SemaphoreType.DMA,
    ],
)(_sc_body)


def _tc_body(x_ref, acc_ref, cnt_ref, wl_ref, wr_ref, b_ref, o_ref):
    mean_cat = jnp.concatenate([acc_ref[0, p] for p in range(NP)], axis=-1)
    cnt = cnt_ref[0, :, :1] + cnt_ref[1, :, :1]
    mean = mean_cat / jnp.maximum(cnt, 1.0)
    o = jnp.dot(mean, wl_ref[...], preferred_element_type=jnp.float32)
    o = o + jnp.dot(x_ref[0], wr_ref[...], preferred_element_type=jnp.float32)
    o = o + b_ref[...]
    o_ref[0] = jnp.maximum(o, 0.0)


def _tc_tail(x, acc, cnt, W_l, W_r, b2):
    grid = (NB, NN // BLK)
    return pl.pallas_call(
        _tc_body,
        grid=grid,
        in_specs=[
            pl.BlockSpec((1, BLK, D), lambda i, j: (i, j, 0)),
            pl.BlockSpec((1, NP, BLK, CW), lambda i, j: (i, 0, j, 0)),
            pl.BlockSpec((NC, BLK, CW), lambda i, j: (0, j, 0)),
            pl.BlockSpec((D, D), lambda i, j: (0, 0)),
            pl.BlockSpec((D, D), lambda i, j: (0, 0)),
            pl.BlockSpec((1, D), lambda i, j: (0, 0)),
        ],
        out_specs=pl.BlockSpec((1, BLK, D), lambda i, j: (i, j, 0)),
        out_shape=jax.ShapeDtypeStruct((NB, NN, D), jnp.float32),
    )(x, acc, cnt, W_l, W_r, b2)


def kernel(inputs, adj, W_l, W_r, b):
    x = inputs
    xr = x.reshape(NB, NN, NP, CW).transpose(0, 2, 1, 3).reshape(-1, CW)
    src = adj[0]
    srcb = (src[None, :] + (jnp.arange(NG, dtype=jnp.int32) * NN)[:, None]
            ).reshape(NG, NS, NCHUNK, CHUNK)
    dstb = adj[1].reshape(NS, NCHUNK, CHUNK)
    acc, cnt = _sc_segment_sum(srcb, dstb, xr)
    return _tc_tail(x, acc, cnt, W_l, W_r, b.reshape(1, D))


# speedup vs baseline: 5.2817x; 1.5311x over previous
import functools

import jax
import jax.numpy as jnp
from jax import lax
from jax.experimental import pallas as pl
from jax.experimental.pallas import tpu as pltpu
from jax.experimental.pallas import tpu_sc as plsc

NB = 2
NN = 10000
NE = 320000
D = 128

NC = 2
NS = 16
NW = NC * NS
NP = D // 16
CHUNK = 128
EPT = NE // NS
NCHUNK = 158
EPTP = NCHUNK * CHUNK
NCCHUNK = NCHUNK // NC
NNP = 10240
TRASH = NNP - 1
RPT = NNP // NS
CW = 16
RING = 4
BLK = 2000


def _sc_body(srcb, dstb, xr, acc_out, cnt_out,
             src_st, dst_st, rows_v0, rows_v1, rows_v2, rows_v3,
             ones_v, zbuf_v, acc_sh,
             sg0, sg1, sg2, sg3, ss0, ss1, ss2, ss3):
    rows_v = [rows_v0, rows_v1, rows_v2, rows_v3]
    sg = [sg0, sg1, sg2, sg3]
    ss = [ss0, ss1, ss2, ss3]

    c = lax.axis_index("c")
    s = lax.axis_index("s")
    rbase = s * RPT

    zero16 = jnp.zeros((16,), jnp.float32)
    one16 = jnp.ones((16,), jnp.float32)

    def fill_zbuf(i, _):
        zbuf_v[i, pl.ds(0, 16)] = zero16
        return 0
    lax.fori_loop(0, RPT, fill_zbuf, 0)

    def fill_ones(i, _):
        ones_v[i, pl.ds(0, 16)] = one16
        return 0
    lax.fori_loop(0, CHUNK, fill_ones, 0)

    pltpu.sync_copy(srcb.at[c, s], src_st)
    pltpu.sync_copy(dstb.at[s], dst_st)
    pltpu.sync_copy(zbuf_v, acc_sh.at[pl.ds(rbase, RPT)])
    plsc.subcore_barrier()

    cbase = c * NCCHUNK

    def cnt_scat(j, r):
        pltpu.async_copy(ones_v, acc_sh.at[dst_st.at[cbase + j]],
                         ss[r], add=True)

    def cnt_drain(j, r):
        pltpu.make_async_copy(ones_v, acc_sh.at[dst_st.at[cbase + j]],
                              ss[r]).wait()

    def cnt_visit(j, r, dsc):
        if dsc:
            cnt_drain(j - RING, r)
        cnt_scat(j, r)

    for j in range(8):
        cnt_visit(j, j % RING, j >= RING)

    def cbody(i, _):
        for r in range(RING):
            cnt_visit(RING * i + r, r, True)
        return 0
    lax.fori_loop(2, NCCHUNK // RING, cbody, 0)
    for j in range(RING * (NCCHUNK // RING), NCCHUNK):
        cnt_visit(j, j % RING, True)
    for j in range(NCCHUNK - RING, NCCHUNK):
        cnt_drain(j, j % RING)

    plsc.subcore_barrier()
    pltpu.sync_copy(acc_sh.at[pl.ds(rbase, RPT)],
                    cnt_out.at[c, pl.ds(rbase, RPT)])
    pltpu.sync_copy(zbuf_v, acc_sh.at[pl.ds(rbase, RPT)])
    plsc.subcore_barrier()

    for p in range(NP):
        tab = xr.at[p]

        def fire_gather(j, r):
            pltpu.async_copy(tab.at[src_st.at[j]], rows_v[r], sg[r])

        def fire_scatter(j, r):
            pltpu.make_async_copy(tab.at[pl.ds(0, CHUNK)], rows_v[r],
                                  sg[r]).wait()
            pltpu.async_copy(rows_v[r], acc_sh.at[dst_st.at[j]],
                             ss[r], add=True)

        def drain_scatter(j, r):
            pltpu.make_async_copy(rows_v[r], acc_sh.at[dst_st.at[j]],
                                  ss[r]).wait()

        def visit(j, r, dsc, dscp):
            if dsc:
                drain_scatter(j - RING, r)
            fire_gather(j, r)
            if dscp:
                fire_scatter(j - 3, (r + 1) % RING)

        for j in range(8):
            visit(j, j % RING, j >= RING, j >= 3)

        def body(i, _):
            for r in range(RING):
                visit(RING * i + r, r, True, True)
            return 0
        lax.fori_loop(2, NCHUNK // RING, body, 0)
        for j in range(RING * (NCHUNK // RING), NCHUNK):
            visit(j, j % RING, True, True)
        for j in range(NCHUNK, NCHUNK + 3):
            fire_scatter(j - 3, (j + 1) % RING)
        for j in range(NCHUNK - RING, NCHUNK):
            drain_scatter(j, j % RING)

        plsc.subcore_barrier()
        pltpu.sync_copy(acc_sh.at[pl.ds(rbase, RPT)],
                        acc_out.at[c, p, pl.ds(rbase, RPT)])
        if p + 1 < NP:
            pltpu.sync_copy(zbuf_v, acc_sh.at[pl.ds(rbase, RPT)])
            plsc.subcore_barrier()


_sc_segment_sum = functools.partial(
    pl.kernel,
    out_type=(
        jax.ShapeDtypeStruct((NB, NP, NNP, CW), jnp.float32),
        jax.ShapeDtypeStruct((NC, NNP, CW), jnp.float32),
    ),
    mesh=plsc.VectorSubcoreMesh(core_axis_name="c", subcore_axis_name="s"),
    compiler_params=pltpu.CompilerParams(use_tc_tiling_on_sc=False),
    scratch_types=[
        pltpu.VMEM((NCHUNK, CHUNK), jnp.int32),
        pltpu.VMEM((NCHUNK, CHUNK), jnp.int32),
        pltpu.VMEM((CHUNK, CW), jnp.float32),
        pltpu.VMEM((CHUNK, CW), jnp.float32),
        pltpu.VMEM((CHUNK, CW), jnp.float32),
        pltpu.VMEM((CHUNK, CW), jnp.float32),
        pltpu.VMEM((CHUNK, CW), jnp.float32),
        pltpu.VMEM((RPT, CW), jnp.float32),
        pltpu.VMEM_SHARED((NNP, CW), jnp.float32),
        pltpu.SemaphoreType.DMA,
        pltpu.SemaphoreType.DMA,
        pltpu.SemaphoreType.DMA,
        pltpu.SemaphoreType.DMA,
        pltpu.SemaphoreType.DMA,
        pltpu.SemaphoreType.DMA,
        pltpu.SemaphoreType.DMA,
        pltpu.SemaphoreType.DMA,
    ],
)(_sc_body)


def _tc_body(x_ref, acc_ref, cnt_ref, wl_ref, wr_ref, b_ref, o_ref):
    mean_cat = jnp.concatenate([acc_ref[0, p] for p in range(NP)], axis=-1)
    cnt = cnt_ref[0, :, :1] + cnt_ref[1, :, :1]
    mean = mean_cat / jnp.maximum(cnt, 1.0)
    o = jnp.dot(mean, wl_ref[...], preferred_element_type=jnp.float32)
    o = o + jnp.dot(x_ref[0], wr_ref[...], preferred_element_type=jnp.float32)
    o = o + b_ref[...]
    o_ref[0] = jnp.maximum(o, 0.0)


def _tc_tail(x, acc, cnt, W_l, W_r, b2):
    grid = (NB, NN // BLK)
    return pl.pallas_call(
        _tc_body,
        grid=grid,
        in_specs=[
            pl.BlockSpec((1, BLK, D), lambda i, j: (i, j, 0)),
            pl.BlockSpec((1, NP, BLK, CW), lambda i, j: (i, 0, j, 0)),
            pl.BlockSpec((NC, BLK, CW), lambda i, j: (0, j, 0)),
            pl.BlockSpec((D, D), lambda i, j: (0, 0)),
            pl.BlockSpec((D, D), lambda i, j: (0, 0)),
            pl.BlockSpec((1, D), lambda i, j: (0, 0)),
        ],
        out_specs=pl.BlockSpec((1, BLK, D), lambda i, j: (i, j, 0)),
        out_shape=jax.ShapeDtypeStruct((NB, NN, D), jnp.float32),
    )(x, acc, cnt, W_l, W_r, b2)


def kernel(inputs, adj, W_l, W_r, b):
    x = inputs
    xr = x.reshape(NB, NN, NP, CW).transpose(2, 0, 1, 3).reshape(NP, -1, CW)
    pad = EPTP - EPT
    src2 = jnp.pad(adj[0].reshape(NS, EPT), ((0, 0), (0, pad)))
    dst2 = jnp.pad(adj[1].reshape(NS, EPT), ((0, 0), (0, pad)),
                   constant_values=TRASH)
    srcb = (src2[None] + (jnp.arange(NB, dtype=jnp.int32) * NN)[:, None, None]
            ).reshape(NB, NS, NCHUNK, CHUNK)
    dstb = dst2.reshape(NS, NCHUNK, CHUNK)
    acc, cnt = _sc_segment_sum(srcb, dstb, xr)
    return _tc_tail(x, acc, cnt, W_l, W_r, b.reshape(1, D))


# speedup vs baseline: 5.7817x; 1.0947x over previous
import functools

import jax
import jax.numpy as jnp
from jax import lax
from jax.experimental import pallas as pl
from jax.experimental.pallas import tpu as pltpu
from jax.experimental.pallas import tpu_sc as plsc

NB = 2
NN = 10000
NE = 320000
D = 128

NC = 2
NS = 16
NW = NC * NS
NP = D // 16
CHUNK = 128
EPT = NE // NS
NCHUNK = 158
EPTP = NCHUNK * CHUNK
NCCHUNK = NCHUNK // NC
NNP = 10240
TRASH = NNP - 1
RPT = NNP // NS
CW = 16
RING = 8
LAG = 4
BLK = 2000


def _sc_body(srcb, dstb, xr, acc_out, cnt_out,
             src_st, dst_st,
             rows_v0, rows_v1, rows_v2, rows_v3,
             rows_v4, rows_v5, rows_v6, rows_v7,
             ones_v, zbuf_v, acc_sh,
             sg0, sg1, sg2, sg3, sg4, sg5, sg6, sg7,
             ss0, ss1, ss2, ss3, ss4, ss5, ss6, ss7):
    rows_v = [rows_v0, rows_v1, rows_v2, rows_v3,
              rows_v4, rows_v5, rows_v6, rows_v7]
    sg = [sg0, sg1, sg2, sg3, sg4, sg5, sg6, sg7]
    ss = [ss0, ss1, ss2, ss3, ss4, ss5, ss6, ss7]

    c = lax.axis_index("c")
    s = lax.axis_index("s")
    rbase = s * RPT

    zero16 = jnp.zeros((16,), jnp.float32)
    one16 = jnp.ones((16,), jnp.float32)

    def fill_zbuf(i, _):
        zbuf_v[i, pl.ds(0, 16)] = zero16
        return 0
    lax.fori_loop(0, RPT, fill_zbuf, 0)

    def fill_ones(i, _):
        ones_v[i, pl.ds(0, 16)] = one16
        return 0
    lax.fori_loop(0, CHUNK, fill_ones, 0)

    pltpu.sync_copy(dstb.at[s], dst_st)
    pltpu.sync_copy(zbuf_v, acc_sh.at[pl.ds(rbase, RPT)])
    plsc.subcore_barrier()

    cbase = c * NCCHUNK

    def cnt_scat(j, r):
        pltpu.async_copy(ones_v, acc_sh.at[dst_st.at[cbase + j]],
                         ss[r], add=True)

    def cnt_drain(j, r):
        pltpu.make_async_copy(ones_v, acc_sh.at[dst_st.at[cbase + j]],
                              ss[r]).wait()

    def cnt_visit(j, r, dsc):
        if dsc:
            cnt_drain(j - RING, r)
        cnt_scat(j, r)

    for j in range(2 * RING):
        cnt_visit(j, j % RING, j >= RING)

    def cbody(i, _):
        for r in range(RING):
            cnt_visit(RING * i + r, r, True)
        return 0
    lax.fori_loop(2, NCCHUNK // RING, cbody, 0)
    for j in range(RING * (NCCHUNK // RING), NCCHUNK):
        cnt_visit(j, j % RING, True)
    for j in range(NCCHUNK - RING, NCCHUNK):
        cnt_drain(j, j % RING)

    plsc.subcore_barrier()
    pltpu.sync_copy(acc_sh.at[pl.ds(rbase, RPT)],
                    cnt_out.at[c, pl.ds(rbase, RPT)])
    pltpu.sync_copy(zbuf_v, acc_sh.at[pl.ds(rbase, RPT)])
    plsc.subcore_barrier()

    for p in range(NP):
        pltpu.sync_copy(srcb.at[c * NP + p, s], src_st)

        def fire_gather(j, r):
            pltpu.async_copy(xr.at[src_st.at[j]], rows_v[r], sg[r])

        def fire_scatter(j, r):
            pltpu.make_async_copy(xr.at[pl.ds(0, CHUNK)], rows_v[r],
                                  sg[r]).wait()
            pltpu.async_copy(rows_v[r], acc_sh.at[dst_st.at[j]],
                             ss[r], add=True)

        def drain_scatter(j, r):
            pltpu.make_async_copy(rows_v[r], acc_sh.at[dst_st.at[j]],
                                  ss[r]).wait()

        def visit(j, r, dsc, dscp):
            if dsc:
                drain_scatter(j - RING, r)
            fire_gather(j, r)
            if dscp:
                fire_scatter(j - LAG, (r + LAG) % RING)

        for j in range(2 * RING):
            visit(j, j % RING, j >= RING, j >= LAG)

        def body(i, _):
            for r in range(RING):
                visit(RING * i + r, r, True, True)
            return 0
        lax.fori_loop(2, NCHUNK // RING, body, 0)
        for j in range(RING * (NCHUNK // RING), NCHUNK):
            visit(j, j % RING, True, True)
        for j in range(NCHUNK, NCHUNK + LAG):
            fire_scatter(j - LAG, (j + LAG) % RING)
        for j in range(NCHUNK - RING, NCHUNK):
            drain_scatter(j, j % RING)

        plsc.subcore_barrier()
        pltpu.sync_copy(acc_sh.at[pl.ds(rbase, RPT)],
                        acc_out.at[c, p, pl.ds(rbase, RPT)])
        if p + 1 < NP:
            pltpu.sync_copy(zbuf_v, acc_sh.at[pl.ds(rbase, RPT)])
            plsc.subcore_barrier()


_sc_segment_sum = functools.partial(
    pl.kernel,
    out_type=(
        jax.ShapeDtypeStruct((NB, NP, NNP, CW), jnp.float32),
        jax.ShapeDtypeStruct((NC, NNP, CW), jnp.float32),
    ),
    mesh=plsc.VectorSubcoreMesh(core_axis_name="c", subcore_axis_name="s"),
    compiler_params=pltpu.CompilerParams(use_tc_tiling_on_sc=False),
    scratch_types=[
        pltpu.VMEM((NCHUNK, CHUNK), jnp.int32),
        pltpu.VMEM((NCHUNK, CHUNK), jnp.int32),
        pltpu.VMEM((CHUNK, CW), jnp.float32),
        pltpu.VMEM((CHUNK, CW), jnp.float32),
        pltpu.VMEM((CHUNK, CW), jnp.float32),
        pltpu.VMEM((CHUNK, CW), jnp.float32),
        pltpu.VMEM((CHUNK, CW), jnp.float32),
        pltpu.VMEM((CHUNK, CW), jnp.float32),
        pltpu.VMEM((CHUNK, CW), jnp.float32),
        pltpu.VMEM((CHUNK, CW), jnp.float32),
        pltpu.VMEM((CHUNK, CW), jnp.float32),
        pltpu.VMEM((RPT, CW), jnp.float32),
        pltpu.VMEM_SHARED((NNP, CW), jnp.float32),
        pltpu.SemaphoreType.DMA,
        pltpu.SemaphoreType.DMA,
        pltpu.SemaphoreType.DMA,
        pltpu.SemaphoreType.DMA,
        pltpu.SemaphoreType.DMA,
        pltpu.SemaphoreType.DMA,
        pltpu.SemaphoreType.DMA,
        pltpu.SemaphoreType.DMA,
        pltpu.SemaphoreType.DMA,
        pltpu.SemaphoreType.DMA,
        pltpu.SemaphoreType.DMA,
        pltpu.SemaphoreType.DMA,
        pltpu.SemaphoreType.DMA,
        pltpu.SemaphoreType.DMA,
        pltpu.SemaphoreType.DMA,
        pltpu.SemaphoreType.DMA,
    ],
)(_sc_body)


def _tc_body(x_ref, acc_ref, cnt_ref, wl_ref, wr_ref, b_ref, o_ref):
    mean_cat = jnp.concatenate([acc_ref[0, p] for p in range(NP)], axis=-1)
    cnt = cnt_ref[0, :, :1] + cnt_ref[1, :, :1]
    mean = mean_cat / jnp.maximum(cnt, 1.0)
    o = jnp.dot(mean, wl_ref[...], preferred_element_type=jnp.float32)
    o = o + jnp.dot(x_ref[0], wr_ref[...], preferred_element_type=jnp.float32)
    o = o + b_ref[...]
    o_ref[0] = jnp.maximum(o, 0.0)


def _tc_tail(x, acc, cnt, W_l, W_r, b2):
    grid = (NB, NN // BLK)
    return pl.pallas_call(
        _tc_body,
        grid=grid,
        in_specs=[
            pl.BlockSpec((1, BLK, D), lambda i, j: (i, j, 0)),
            pl.BlockSpec((1, NP, BLK, CW), lambda i, j: (i, 0, j, 0)),
            pl.BlockSpec((NC, BLK, CW), lambda i, j: (0, j, 0)),
            pl.BlockSpec((D, D), lambda i, j: (0, 0)),
            pl.BlockSpec((D, D), lambda i, j: (0, 0)),
            pl.BlockSpec((1, D), lambda i, j: (0, 0)),
        ],
        out_specs=pl.BlockSpec((1, BLK, D), lambda i, j: (i, j, 0)),
        out_shape=jax.ShapeDtypeStruct((NB, NN, D), jnp.float32),
    )(x, acc, cnt, W_l, W_r, b2)


def kernel(inputs, adj, W_l, W_r, b):
    x = inputs
    xr = x.reshape(NB * NN * NP, CW)
    pad = EPTP - EPT
    src2 = jnp.pad(adj[0].reshape(NS, EPT), ((0, 0), (0, pad)))
    dst2 = jnp.pad(adj[1].reshape(NS, EPT), ((0, 0), (0, pad)),
                   constant_values=TRASH)
    boff = (jnp.arange(NB, dtype=jnp.int32) * NN)[:, None, None, None]
    poff = jnp.arange(NP, dtype=jnp.int32)[None, :, None, None]
    srcb = ((src2[None, None] + boff) * NP + poff
            ).reshape(NB * NP, NS, NCHUNK, CHUNK)
    dstb = dst2.reshape(NS, NCHUNK, CHUNK)
    acc, cnt = _sc_segment_sum(srcb, dstb, xr)
    return _tc_tail(x, acc, cnt, W_l, W_r, b.reshape(1, D))


# speedup vs baseline: 6.6420x; 1.1488x over previous
import functools

import jax
import jax.numpy as jnp
from jax import lax
from jax.experimental import pallas as pl
from jax.experimental.pallas import tpu as pltpu
from jax.experimental.pallas import tpu_sc as plsc

NB = 2
NN = 10000
NE = 320000
D = 128

NC = 2
NS = 16
NW = NC * NS
NP = D // 16
CHUNK = 128
EPT = NE // NS
NCHUNK = 158
EPTP = NCHUNK * CHUNK
NCCHUNK = NCHUNK // NC
NNP = 10240
TRASH = NNP - 1
RPT = NNP // NS
CW = 16
RING = 8
LAG = 4
BLK = 2000


def _sc_body(srcp, dstb, xr, acc_out, cnt_out,
             src_raw, dst_st, idx_a, idx_b,
             rows_v0, rows_v1, rows_v2, rows_v3,
             rows_v4, rows_v5, rows_v6, rows_v7,
             ones_v, zbuf_v, acc_sh,
             sg0, sg1, sg2, sg3, sg4, sg5, sg6, sg7,
             ss0, ss1, ss2, ss3, ss4, ss5, ss6, ss7):
    rows_v = [rows_v0, rows_v1, rows_v2, rows_v3,
              rows_v4, rows_v5, rows_v6, rows_v7]
    sg = [sg0, sg1, sg2, sg3, sg4, sg5, sg6, sg7]
    ss = [ss0, ss1, ss2, ss3, ss4, ss5, ss6, ss7]

    c = lax.axis_index("c")
    s = lax.axis_index("s")
    rbase = s * RPT

    zero16 = jnp.zeros((16,), jnp.float32)
    one16 = jnp.ones((16,), jnp.float32)

    def fill_zbuf(i, _):
        zbuf_v[i, pl.ds(0, 16)] = zero16
        return 0
    lax.fori_loop(0, RPT, fill_zbuf, 0)

    def fill_ones(i, _):
        ones_v[i, pl.ds(0, 16)] = one16
        return 0
    lax.fori_loop(0, CHUNK, fill_ones, 0)

    pltpu.sync_copy(srcp.at[s], src_raw)
    pltpu.sync_copy(dstb.at[s], dst_st)

    idx_buf = [idx_a, idx_b]

    def xform(j, pnext, buf):
        off = c * (NN * NP) + pnext
        for k in range(CHUNK // 16):
            sl = pl.ds(k * 16, 16)
            buf[j, sl] = src_raw[j, sl] * NP + off
    pltpu.sync_copy(zbuf_v, acc_sh.at[pl.ds(rbase, RPT)])
    plsc.subcore_barrier()

    cbase = c * NCCHUNK

    def cnt_scat(j, r):
        pltpu.async_copy(ones_v, acc_sh.at[dst_st.at[cbase + j]],
                         ss[r], add=True)

    def cnt_drain(j, r):
        pltpu.make_async_copy(ones_v, acc_sh.at[dst_st.at[cbase + j]],
                              ss[r]).wait()

    def cnt_visit(j, r, dsc):
        if dsc:
            cnt_drain(j - RING, r)
        cnt_scat(j, r)
        xform(2 * j, 0, idx_a)
        xform(2 * j + 1, 0, idx_a)

    for j in range(2 * RING):
        cnt_visit(j, j % RING, j >= RING)

    def cbody(i, _):
        for r in range(RING):
            cnt_visit(RING * i + r, r, True)
        return 0
    lax.fori_loop(2, NCCHUNK // RING, cbody, 0)
    for j in range(RING * (NCCHUNK // RING), NCCHUNK):
        cnt_visit(j, j % RING, True)
    for j in range(NCCHUNK - RING, NCCHUNK):
        cnt_drain(j, j % RING)

    plsc.subcore_barrier()
    pltpu.sync_copy(acc_sh.at[pl.ds(rbase, RPT)],
                    cnt_out.at[c, pl.ds(rbase, RPT)])
    pltpu.sync_copy(zbuf_v, acc_sh.at[pl.ds(rbase, RPT)])
    plsc.subcore_barrier()

    for p in range(NP):
        cur = idx_buf[p % 2]
        nxt = idx_buf[(p + 1) % 2]

        def fire_gather(j, r):
            pltpu.async_copy(xr.at[cur.at[j]], rows_v[r], sg[r])

        def fire_scatter(j, r):
            pltpu.make_async_copy(xr.at[pl.ds(0, CHUNK)], rows_v[r],
                                  sg[r]).wait()
            pltpu.async_copy(rows_v[r], acc_sh.at[dst_st.at[j]],
                             ss[r], add=True)

        def drain_scatter(j, r):
            pltpu.make_async_copy(rows_v[r], acc_sh.at[dst_st.at[j]],
                                  ss[r]).wait()

        def visit(j, r, dsc, dscp):
            if dsc:
                drain_scatter(j - RING, r)
            fire_gather(j, r)
            if dscp:
                fire_scatter(j - LAG, (r + LAG) % RING)
            if p + 1 < NP:
                xform(j, p + 1, nxt)

        for j in range(2 * RING):
            visit(j, j % RING, j >= RING, j >= LAG)

        def body(i, _):
            for r in range(RING):
                visit(RING * i + r, r, True, True)
            return 0
        lax.fori_loop(2, NCHUNK // RING, body, 0)
        for j in range(RING * (NCHUNK // RING), NCHUNK):
            visit(j, j % RING, True, True)
        for j in range(NCHUNK, NCHUNK + LAG):
            fire_scatter(j - LAG, (j + LAG) % RING)
        for j in range(NCHUNK - RING, NCHUNK):
            drain_scatter(j, j % RING)

        plsc.subcore_barrier()
        pltpu.sync_copy(acc_sh.at[pl.ds(rbase, RPT)],
                        acc_out.at[c, p, pl.ds(rbase, RPT)])
        if p + 1 < NP:
            pltpu.sync_copy(zbuf_v, acc_sh.at[pl.ds(rbase, RPT)])
            plsc.subcore_barrier()


_sc_segment_sum = functools.partial(
    pl.kernel,
    out_type=(
        jax.ShapeDtypeStruct((NB, NP, NNP, CW), jnp.float32),
        jax.ShapeDtypeStruct((NC, NNP, CW), jnp.float32),
    ),
    mesh=plsc.VectorSubcoreMesh(core_axis_name="c", subcore_axis_name="s"),
    compiler_params=pltpu.CompilerParams(use_tc_tiling_on_sc=False),
    scratch_types=[
        pltpu.VMEM((NCHUNK, CHUNK), jnp.int32),
        pltpu.VMEM((NCHUNK, CHUNK), jnp.int32),
        pltpu.VMEM((NCHUNK, CHUNK), jnp.int32),
        pltpu.VMEM((NCHUNK, CHUNK), jnp.int32),
        pltpu.VMEM((CHUNK, CW), jnp.float32),
        pltpu.VMEM((CHUNK, CW), jnp.float32),
        pltpu.VMEM((CHUNK, CW), jnp.float32),
        pltpu.VMEM((CHUNK, CW), jnp.float32),
        pltpu.VMEM((CHUNK, CW), jnp.float32),
        pltpu.VMEM((CHUNK, CW), jnp.float32),
        pltpu.VMEM((CHUNK, CW), jnp.float32),
        pltpu.VMEM((CHUNK, CW), jnp.float32),
        pltpu.VMEM((CHUNK, CW), jnp.float32),
        pltpu.VMEM((RPT, CW), jnp.float32),
        pltpu.VMEM_SHARED((NNP, CW), jnp.float32),
        pltpu.SemaphoreType.DMA,
        pltpu.SemaphoreType.DMA,
        pltpu.SemaphoreType.DMA,
        pltpu.SemaphoreType.DMA,
        pltpu.SemaphoreType.DMA,
        pltpu.SemaphoreType.DMA,
        pltpu.SemaphoreType.DMA,
        pltpu.SemaphoreType.DMA,
        pltpu.SemaphoreType.DMA,
        pltpu.SemaphoreType.DMA,
        pltpu.SemaphoreType.DMA,
        pltpu.SemaphoreType.DMA,
        pltpu.SemaphoreType.DMA,
        pltpu.SemaphoreType.DMA,
        pltpu.SemaphoreType.DMA,
        pltpu.SemaphoreType.DMA,
    ],
)(_sc_body)


def _tc_body(x_ref, acc_ref, cnt_ref, wl_ref, wr_ref, b_ref, o_ref):
    mean_cat = jnp.concatenate([acc_ref[0, p] for p in range(NP)], axis=-1)
    cnt = cnt_ref[0, :, :1] + cnt_ref[1, :, :1]
    mean = mean_cat / jnp.maximum(cnt, 1.0)
    o = jnp.dot(mean, wl_ref[...], preferred_element_type=jnp.float32)
    o = o + jnp.dot(x_ref[0], wr_ref[...], preferred_element_type=jnp.float32)
    o = o + b_ref[...]
    o_ref[0] = jnp.maximum(o, 0.0)


def _tc_tail(x, acc, cnt, W_l, W_r, b2):
    grid = (NB, NN // BLK)
    return pl.pallas_call(
        _tc_body,
        grid=grid,
        in_specs=[
            pl.BlockSpec((1, BLK, D), lambda i, j: (i, j, 0)),
            pl.BlockSpec((1, NP, BLK, CW), lambda i, j: (i, 0, j, 0)),
            pl.BlockSpec((NC, BLK, CW), lambda i, j: (0, j, 0)),
            pl.BlockSpec((D, D), lambda i, j: (0, 0)),
            pl.BlockSpec((D, D), lambda i, j: (0, 0)),
            pl.BlockSpec((1, D), lambda i, j: (0, 0)),
        ],
        out_specs=pl.BlockSpec((1, BLK, D), lambda i, j: (i, j, 0)),
        out_shape=jax.ShapeDtypeStruct((NB, NN, D), jnp.float32),
    )(x, acc, cnt, W_l, W_r, b2)


def kernel(inputs, adj, W_l, W_r, b):
    x = inputs
    xr = x.reshape(NB * NN * NP, CW)
    pad = EPTP - EPT
    src2 = jnp.pad(adj[0].reshape(NS, EPT), ((0, 0), (0, pad)))
    dst2 = jnp.pad(adj[1].reshape(NS, EPT), ((0, 0), (0, pad)),
                   constant_values=TRASH)
    srcp = src2.reshape(NS, NCHUNK, CHUNK)
    dstb = dst2.reshape(NS, NCHUNK, CHUNK)
    acc, cnt = _sc_segment_sum(srcp, dstb, xr)
    return _tc_tail(x, acc, cnt, W_l, W_r, b.reshape(1, D))
